# baseline (speedup 1.0000x reference)
import functools

import jax
import jax.numpy as jnp
from jax import lax
from jax.experimental import pallas as pl
from jax.experimental.pallas import tpu as pltpu
from jax.experimental.pallas import tpu_sc as plsc

BATCH = 4096
SEQ = 50
DIM = 64
B = BATCH * SEQ
NC = 2
NS = 16
NW = NC * NS
RPW = BATCH // NW
BPW = RPW * SEQ
CH = SEQ
NCH = RPW
G = 8
NGROUPS = NCH // G

_mesh = plsc.VectorSubcoreMesh(
    core_axis_name="c", subcore_axis_name="s", num_cores=NC, num_subcores=NS
)


@functools.partial(
    pl.kernel,
    out_type=jax.ShapeDtypeStruct((B, DIM), jnp.float32),
    mesh=_mesh,
    scratch_types=[
        pltpu.VMEM((NCH, CH), jnp.int32),
        pltpu.VMEM((2 * G, CH, DIM), jnp.float32),
        pltpu.SemaphoreType.DMA,
        pltpu.SemaphoreType.DMA,
    ],
    compiler_params=pltpu.CompilerParams(use_tc_tiling_on_sc=False),
)
def _emb_lookup(idx_hbm, table_hbm, out_hbm, idx_v, rows_v, gsem, ssem):
    wid = lax.axis_index("s") * NC + lax.axis_index("c")
    base = wid * BPW
    pltpu.sync_copy(idx_hbm.at[pl.ds(wid * RPW, RPW)], idx_v)

    for b in range(G):
        pltpu.async_copy(table_hbm.at[idx_v.at[b]], rows_v.at[b], gsem)

    @pl.loop(0, NGROUPS)
    def _(k):
        bank = lax.rem(k, 2) * G
        nbank = G - bank

        for b in range(G):
            pltpu.make_async_copy(
                table_hbm.at[idx_v.at[0]], rows_v.at[0], gsem
            ).wait()

        @pl.when(k >= 1)
        def _():
            for b in range(G):
                pltpu.make_async_copy(
                    rows_v.at[0], out_hbm.at[pl.ds(base, CH)], ssem
                ).wait()

        @pl.when(k + 1 < NGROUPS)
        def _():
            for b in range(G):
                pltpu.async_copy(
                    table_hbm.at[idx_v.at[(k + 1) * G + b]],
                    rows_v.at[nbank + b],
                    gsem,
                )

        for b in range(G):
            pltpu.async_copy(
                rows_v.at[bank + b],
                out_hbm.at[pl.ds(base + (k * G + b) * CH, CH)],
                ssem,
            )

    for b in range(G):
        pltpu.make_async_copy(
            rows_v.at[0], out_hbm.at[pl.ds(base, CH)], ssem
        ).wait()


def kernel(token_ids, table):
    out = _emb_lookup(token_ids.astype(jnp.int32), table)
    return out.reshape(BATCH, SEQ, DIM)


# speedup vs baseline: 1.0151x; 1.0151x over previous
import functools

import jax
import jax.numpy as jnp
from jax import lax
from jax.experimental import pallas as pl
from jax.experimental.pallas import tpu as pltpu
from jax.experimental.pallas import tpu_sc as plsc

BATCH = 4096
SEQ = 50
DIM = 64
B = BATCH * SEQ
NC = 2
NS = 16
NW = NC * NS
CPW = BATCH // NW
CH = CPW
G = 5
NGROUPS = SEQ // G

_mesh = plsc.VectorSubcoreMesh(
    core_axis_name="c", subcore_axis_name="s", num_cores=NC, num_subcores=NS
)


@functools.partial(
    pl.kernel,
    out_type=jax.ShapeDtypeStruct((B, DIM), jnp.float32),
    mesh=_mesh,
    scratch_types=[
        pltpu.VMEM((SEQ, CH), jnp.int32),
        pltpu.VMEM((2 * G, CH, DIM), jnp.float32),
        pltpu.SemaphoreType.DMA,
        pltpu.SemaphoreType.DMA,
    ],
    compiler_params=pltpu.CompilerParams(use_tc_tiling_on_sc=False),
)
def _emb_lookup(idx_hbm, table_hbm, out_hbm, idx_v, rows_v, gsem, ssem):
    wid = lax.axis_index("s") * NC + lax.axis_index("c")
    col = wid * CPW
    pltpu.sync_copy(idx_hbm.at[:, pl.ds(col, CPW)], idx_v)

    for b in range(G):
        pltpu.async_copy(table_hbm.at[idx_v.at[b]], rows_v.at[b], gsem)

    @pl.loop(0, NGROUPS)
    def _(k):
        bank = lax.rem(k, 2) * G
        nbank = G - bank

        for b in range(G):
            pltpu.make_async_copy(
                table_hbm.at[idx_v.at[0]], rows_v.at[0], gsem
            ).wait()

        @pl.when(k >= 1)
        def _():
            for b in range(G):
                pltpu.make_async_copy(
                    rows_v.at[0], out_hbm.at[pl.ds(col, CH)], ssem
                ).wait()

        @pl.when(k + 1 < NGROUPS)
        def _():
            for b in range(G):
                pltpu.async_copy(
                    table_hbm.at[idx_v.at[(k + 1) * G + b]],
                    rows_v.at[nbank + b],
                    gsem,
                )

        for b in range(G):
            pltpu.async_copy(
                rows_v.at[bank + b],
                out_hbm.at[pl.ds((k * G + b) * BATCH + col, CH)],
                ssem,
            )

    for b in range(G):
        pltpu.make_async_copy(
            rows_v.at[0], out_hbm.at[pl.ds(col, CH)], ssem
        ).wait()


def kernel(token_ids, table):
    idx_t = token_ids.T.astype(jnp.int32)
    out = _emb_lookup(idx_t, table)
    return out.reshape(SEQ, BATCH, DIM).transpose(1, 0, 2)
